# baseline (speedup 1.0000x reference)
import jax
import jax.numpy as jnp
from jax.experimental import pallas as pl
from jax.experimental.pallas import tpu as pltpu

_E = 8
_K = 2
_T = 32768
_D = 1024
_TILE = 1024


def _fused_body(x_ref, w_ref, y_ref, hist_ref):
    i = pl.program_id(0)
    x = x_ref[...]
    w = w_ref[...]
    logits = jax.lax.dot_general(
        x, w, (((1,), (0,)), ((), ())), preferred_element_type=jnp.float32
    )

    iota_e = jax.lax.broadcasted_iota(jnp.int32, logits.shape, 1)
    v0 = jnp.max(logits, axis=-1, keepdims=True)
    first = jnp.min(jnp.where(logits == v0, iota_e, _E), axis=-1, keepdims=True)
    masked = jnp.where(iota_e == first, -jnp.inf, logits)
    v1 = jnp.max(masked, axis=-1, keepdims=True)
    second = jnp.min(jnp.where(masked == v1, iota_e, _E), axis=-1, keepdims=True)

    e1 = jnp.exp(v1 - v0)
    s = 1.0 + e1
    wsum = 1.0 / s + e1 / s
    y_ref[...] = x * wsum

    @pl.when(i == 0)
    def _init():
        hist_ref[...] = jnp.zeros_like(hist_ref)

    iota_l = jax.lax.broadcasted_iota(jnp.int32, (x.shape[0], 128), 1)
    onehot2 = (iota_l == first).astype(jnp.int32) + (iota_l == second).astype(
        jnp.int32
    )
    hist_ref[...] += jnp.sum(onehot2, axis=0, keepdims=True)


def kernel(x, router_weight):
    grid = (_T // _TILE,)
    combined, hist = pl.pallas_call(
        _fused_body,
        grid=grid,
        in_specs=[
            pl.BlockSpec((_TILE, _D), lambda i: (i, 0)),
            pl.BlockSpec((_D, _E), lambda i: (0, 0)),
        ],
        out_specs=[
            pl.BlockSpec((_TILE, _D), lambda i: (i, 0)),
            pl.BlockSpec((1, 128), lambda i: (0, 0)),
        ],
        out_shape=[
            jax.ShapeDtypeStruct((_T, _D), jnp.float32),
            jax.ShapeDtypeStruct((1, 128), jnp.int32),
        ],
        compiler_params=pltpu.CompilerParams(
            dimension_semantics=("arbitrary",),
        ),
    )(x, router_weight)
    return combined, hist[0, :_E]


# speedup vs baseline: 1.0959x; 1.0959x over previous
import jax
import jax.numpy as jnp
from jax.experimental import pallas as pl
from jax.experimental.pallas import tpu as pltpu

_E = 8
_K = 2
_T = 32768
_D = 1024
_TILE = 2048


def _fused_body(x_ref, w_ref, y_ref, hist_ref):
    x = x_ref[...]
    w = w_ref[...]
    logits = jax.lax.dot_general(
        x, w, (((1,), (0,)), ((), ())), preferred_element_type=jnp.float32
    )

    iota_e = jax.lax.broadcasted_iota(jnp.int32, logits.shape, 1)
    v0 = jnp.max(logits, axis=-1, keepdims=True)
    first = jnp.min(jnp.where(logits == v0, iota_e, _E), axis=-1, keepdims=True)
    masked = jnp.where(iota_e == first, -jnp.inf, logits)
    v1 = jnp.max(masked, axis=-1, keepdims=True)
    second = jnp.min(jnp.where(masked == v1, iota_e, _E), axis=-1, keepdims=True)

    e1 = jnp.exp(v1 - v0)
    s = 1.0 + e1
    wsum = 1.0 / s + e1 / s
    y_ref[...] = x * wsum

    iota_l = jax.lax.broadcasted_iota(jnp.int32, (x.shape[0], 128), 1)
    onehot2 = (iota_l == first).astype(jnp.int32) + (iota_l == second).astype(
        jnp.int32
    )
    hist_ref[0, ...] = jnp.sum(onehot2, axis=0, keepdims=True)


def kernel(x, router_weight):
    grid = (_T // _TILE,)
    combined, hist = pl.pallas_call(
        _fused_body,
        grid=grid,
        in_specs=[
            pl.BlockSpec((_TILE, _D), lambda i: (i, 0)),
            pl.BlockSpec((_D, _E), lambda i: (0, 0)),
        ],
        out_specs=[
            pl.BlockSpec((_TILE, _D), lambda i: (i, 0)),
            pl.BlockSpec((1, 1, 128), lambda i: (i, 0, 0)),
        ],
        out_shape=[
            jax.ShapeDtypeStruct((_T, _D), jnp.float32),
            jax.ShapeDtypeStruct((grid[0], 1, 128), jnp.int32),
        ],
        compiler_params=pltpu.CompilerParams(
            dimension_semantics=("parallel",),
        ),
    )(x, router_weight)
    return combined, jnp.sum(hist[:, 0, :_E], axis=0)
